# baseline (speedup 1.0000x reference)
import functools

import jax
import jax.numpy as jnp
from jax import lax
from jax.experimental import pallas as pl
from jax.experimental.pallas import tpu as pltpu
from jax.experimental.pallas import tpu_sc as plsc

_NC = 2
_NS = 16
_NW = _NC * _NS
_B = 16384
_D = 64
_BPW = _B // _NW
_CHUNK = 128
_NCHUNK = _BPW // _CHUNK
_L = 16


def _build():
    mesh = plsc.VectorSubcoreMesh(core_axis_name="c", subcore_axis_name="s")

    @functools.partial(
        pl.kernel,
        out_type=jax.ShapeDtypeStruct((_B,), jnp.float32),
        mesh=mesh,
        scratch_types=[
            pltpu.VMEM((_NCHUNK, _CHUNK), jnp.int32),
            pltpu.VMEM((_NCHUNK, _CHUNK), jnp.int32),
            pltpu.VMEM((_BPW, _D), jnp.float32),
            pltpu.VMEM((_BPW, _D), jnp.float32),
            pltpu.VMEM((_BPW,), jnp.float32),
            pltpu.SemaphoreType.DMA,
        ],
        compiler_params=pltpu.CompilerParams(
            needs_layout_passes=False, use_tc_tiling_on_sc=False
        ),
    )
    def run(user_h, item_h, ut_h, it_h, out_h, uidx, iidx, urows, irows, outv, sem):
        wid = lax.axis_index("s") * _NC + lax.axis_index("c")
        base = wid * _BPW

        for j in range(_NCHUNK):
            pltpu.sync_copy(user_h.at[pl.ds(base + j * _CHUNK, _CHUNK)], uidx.at[j])
            pltpu.sync_copy(item_h.at[pl.ds(base + j * _CHUNK, _CHUNK)], iidx.at[j])

        copies = []
        for j in range(_NCHUNK):
            copies.append(
                pltpu.async_copy(ut_h.at[uidx.at[j]], urows.at[pl.ds(j * _CHUNK, _CHUNK)], sem)
            )
            copies.append(
                pltpu.async_copy(it_h.at[iidx.at[j]], irows.at[pl.ds(j * _CHUNK, _CHUNK)], sem)
            )
        for cp in copies:
            cp.wait()

        lanes = lax.iota(jnp.int32, _L)

        def group(g, carry):
            ridx = g * _L + lanes
            acc = jnp.zeros((_L,), jnp.float32)
            for d in range(_D):
                cidx = jnp.full((_L,), d, jnp.int32)
                u16 = plsc.load_gather(urows, [ridx, cidx])
                v16 = plsc.load_gather(irows, [ridx, cidx])
                acc = acc + u16 * v16
            outv[pl.ds(g * _L, _L)] = acc
            return carry

        lax.fori_loop(0, _BPW // _L, group, 0)

        pltpu.sync_copy(outv, out_h.at[pl.ds(base, _BPW)])

    return run


_KERNEL = _build()


def kernel(user, item, user_table, item_table):
    return _KERNEL(
        user.astype(jnp.int32),
        item.astype(jnp.int32),
        user_table,
        item_table,
    )


# speedup vs baseline: 1.1794x; 1.1794x over previous
import functools

import jax
import jax.numpy as jnp
from jax import lax
from jax.experimental import pallas as pl
from jax.experimental.pallas import tpu as pltpu
from jax.experimental.pallas import tpu_sc as plsc

_NC = 2
_NS = 16
_NW = _NC * _NS
_B = 16384
_D = 64
_BPW = _B // _NW
_L = 16
_CHUNK = 128
_NCHUNK = _BPW // _CHUNK


def _build():
    mesh = plsc.VectorSubcoreMesh(core_axis_name="c", subcore_axis_name="s")

    @functools.partial(
        pl.kernel,
        out_type=jax.ShapeDtypeStruct((_B,), jnp.float32),
        mesh=mesh,
        scratch_types=[
            pltpu.VMEM((_NCHUNK, _CHUNK), jnp.int32),
            pltpu.VMEM((_NCHUNK, _CHUNK), jnp.int32),
            pltpu.VMEM((_BPW, _D), jnp.float32),
            pltpu.VMEM((_BPW, _D), jnp.float32),
            pltpu.VMEM((_L, 17), jnp.float32),
            pltpu.VMEM((_BPW,), jnp.float32),
            pltpu.SemaphoreType.DMA,
        ],
        compiler_params=pltpu.CompilerParams(
            needs_layout_passes=False, use_tc_tiling_on_sc=False
        ),
    )
    def run(user_h, item_h, ut_h, it_h, out_h, uidx, iidx, urows, irows, smat,
            outv, sem):
        wid = lax.axis_index("s") * _NC + lax.axis_index("c")
        base = wid * _BPW

        for j in range(_NCHUNK):
            pltpu.sync_copy(user_h.at[pl.ds(base + j * _CHUNK, _CHUNK)], uidx.at[j])
            pltpu.sync_copy(item_h.at[pl.ds(base + j * _CHUNK, _CHUNK)], iidx.at[j])

        handles = []
        for j in range(_NCHUNK):
            cu = pltpu.async_copy(
                ut_h.at[uidx.at[j]], urows.at[pl.ds(j * _CHUNK, _CHUNK)], sem
            )
            ci = pltpu.async_copy(
                it_h.at[iidx.at[j]], irows.at[pl.ds(j * _CHUNK, _CHUNK)], sem
            )
            handles.append((cu, ci))

        lanes = lax.iota(jnp.int32, _L)

        def group(g):
            rbase = g * _L
            for r in range(_L):
                s = None
                for c in range(_D // _L):
                    u = urows[rbase + r, pl.ds(c * _L, _L)]
                    v = irows[rbase + r, pl.ds(c * _L, _L)]
                    s = u * v if s is None else s + u * v
                smat[r, pl.ds(0, _L)] = s
            acc = jnp.zeros((_L,), jnp.float32)
            for j in range(_L):
                col = plsc.load_gather(smat, [lanes, jnp.full((_L,), j, jnp.int32)])
                acc = acc + col
            outv[pl.ds(rbase, _L)] = acc

        for j in range(_NCHUNK):
            cu, ci = handles[j]
            cu.wait()
            ci.wait()

            def chunk_body(gg, carry, j=j):
                group(j * (_CHUNK // _L) + gg)
                return carry

            lax.fori_loop(0, _CHUNK // _L, chunk_body, 0)

        pltpu.sync_copy(outv, out_h.at[pl.ds(base, _BPW)])

    return run


_KERNEL = _build()


def kernel(user, item, user_table, item_table):
    return _KERNEL(
        user.astype(jnp.int32),
        item.astype(jnp.int32),
        user_table,
        item_table,
    )
